# baseline (speedup 1.0000x reference)
import functools

import jax
import jax.numpy as jnp
from jax import lax
from jax.experimental import pallas as pl
from jax.experimental.pallas import tpu as pltpu
from jax.experimental.pallas import tpu_sc as plsc

_B = 128
_NG = 64
_NN = 2 * _NG
_BIGC = 99999.0
_MASKV = 1e15

_LAP_CHUNK = 64


def _first2(mask, iota64, g):
    a = jnp.min(jnp.where(mask != 0, iota64, 64), axis=1, keepdims=True)
    b = jnp.min(jnp.where((mask != 0) & (iota64 != a), iota64, 64),
                axis=1, keepdims=True)
    return a, b


def _lap_body(main_ref, dele_ref, inse_ref, out_ref, work_ref):
    g = main_ref.shape[0]
    work_ref[...] = main_ref[...]
    iota4096 = jax.lax.broadcasted_iota(jnp.int32, (g, 4096), 1)
    mrow = iota4096 >> 6
    mcol = iota4096 & 63
    iota64 = jax.lax.broadcasted_iota(jnp.int32, (g, _NG), 1)

    ones64 = jnp.ones((g, _NG), jnp.int32)
    neg1 = jnp.full((g, 1), -1, jnp.int32)
    dele0 = dele_ref[...]
    inse0 = inse_ref[...]
    init = (jnp.zeros((g, 1), jnp.float32),
            ones64, ones64, ones64, ones64,
            neg1, neg1,
            jnp.zeros((g, 1), jnp.int32))

    def _prefix_sum(x):
        for sh in (1, 2, 4, 8, 16, 32):
            x = x + jnp.concatenate(
                [jnp.zeros((g, sh), jnp.int32), x[:, :_NG - sh]], axis=1)
        return x

    def cond(carry):
        return jnp.any(carry[7] < _NN)

    def step(carry):
        acc, frT, frB, frL, frR, rp, cp, npick = carry
        dele = dele0
        inse = inse0

        work = jnp.where((mrow == rp) | (mcol == cp), _MASKV, work_ref[...])
        work_ref[...] = work
        vm = jnp.min(work, axis=1, keepdims=True)
        km = jnp.min(jnp.where(work == vm, iota4096, 4096),
                     axis=1, keepdims=True)
        fm = ((km >> 6) << 7) + (km & 63)
        de = jnp.where((frT & frR) != 0, dele, _MASKV)
        vd = jnp.min(de, axis=1, keepdims=True)
        rd = jnp.min(jnp.where(de == vd, iota64, 64), axis=1, keepdims=True)
        fd = (rd << 7) + 64 + rd
        ie = jnp.where((frB & frL) != 0, inse, _MASKV)
        vi = jnp.min(ie, axis=1, keepdims=True)
        ji = jnp.min(jnp.where(ie == vi, iota64, 64), axis=1, keepdims=True)
        fi = ((ji + 64) << 7) + ji
        jb0, jb1 = _first2(frB, iota64, g)
        ir0, ir1 = _first2(frR, iota64, g)
        z_ok = (jb0 < 64) & (ir0 < 64)
        vz = jnp.where(z_ok, 0.0, _MASKV)
        fz = ((jb0 + 64) << 7) + 64 + ir0
        rt0, rt1 = _first2(frT, iota64, g)
        useA = (ir1 < 64) | ((ir0 < 64) & (ir0 != rt0))
        rb1 = jnp.where(useA, rt0, rt1)
        ib1 = jnp.where(ir0 != rb1, ir0, ir1)
        b1_ok = (rb1 < 64) & (ib1 < 64)
        vb1 = jnp.where(b1_ok, _BIGC, _MASKV)
        fb1 = (rb1 << 7) + 64 + ib1
        cl0, cl1 = _first2(frL, iota64, g)
        useB = (cl1 < 64) | ((cl0 < 64) & (cl0 != jb0))
        jb2 = jnp.where(useB, jb0, jb1)
        cb2 = jnp.where(cl0 != jb2, cl0, cl1)
        b2_ok = (jb2 < 64) & (cb2 < 64)
        vb2 = jnp.where(b2_ok, _BIGC, _MASKV)
        fb2 = ((jb2 + 64) << 7) + cb2

        best_v, best_f = vm, fm
        for v2, f2 in ((vd, fd), (vi, fi), (vz, fz), (vb1, fb1), (vb2, fb2)):
            take = (v2 < best_v) | ((v2 == best_v) & (f2 < best_f))
            best_v = jnp.where(take, v2, best_v)
            best_f = jnp.where(take, f2, best_f)

        r = best_f >> 7
        c = best_f & 127
        active = npick < _NN
        acc = acc + jnp.where(active, best_v, 0.0)
        batch = (best_f == fz) & (vm > 0.0) & (vd > 0.0) & (vi > 0.0)
        cntB = jnp.sum(frB, axis=1, keepdims=True)
        cntR = jnp.sum(frR, axis=1, keepdims=True)
        cnt = jnp.minimum(cntB, cntR)
        rankB = _prefix_sum(frB)
        rankR = _prefix_sum(frR)
        frB_b = jnp.where(rankB <= cnt, 0, frB)
        frR_b = jnp.where(rankR <= cnt, 0, frR)
        frT = jnp.where(iota64 == r, 0, frT)
        frL = jnp.where(iota64 == c, 0, frL)
        frB = jnp.where(batch, frB_b, jnp.where(iota64 == r - 64, 0, frB))
        frR = jnp.where(batch, frR_b, jnp.where(iota64 == c - 64, 0, frR))
        npick = npick + jnp.where(active, jnp.where(batch, cnt, 1), 0)
        return (acc, frT, frB, frL, frR, r, c, npick)

    acc = jax.lax.while_loop(cond, step, init)[0]
    out_ref[...] = acc


def _lap_mcost(main_flat, dele_diag, inse_diag):
    m = main_flat.shape[0]
    nblk = m // _LAP_CHUNK
    out = pl.pallas_call(
        _lap_body,
        grid=(nblk,),
        in_specs=[pl.BlockSpec((_LAP_CHUNK, _NG * _NG), lambda i: (i, 0)),
                  pl.BlockSpec((_LAP_CHUNK, _NG), lambda i: (i, 0)),
                  pl.BlockSpec((_LAP_CHUNK, _NG), lambda i: (i, 0))],
        out_specs=pl.BlockSpec((_LAP_CHUNK, 1), lambda i: (i, 0)),
        out_shape=jax.ShapeDtypeStruct((m, 1), jnp.float32),
        scratch_shapes=[pltpu.VMEM((_LAP_CHUNK, _NG * _NG), jnp.float32)],
    )(main_flat, dele_diag, inse_diag)
    return out.reshape(m)


_N = 8192
_E = 131072
_NTILE = 16
_ROWS_PER_TILE = _N // _NTILE
_CHUNK = 128
_NCHUNK = _E // _NTILE // _CHUNK


def _sc_mesh():
    return plsc.VectorSubcoreMesh(core_axis_name="c", subcore_axis_name="s")


def _sc_degree(dst_idx, zeros16, ones16):
    @functools.partial(
        pl.kernel, mesh=_sc_mesh(),
        out_type=jax.ShapeDtypeStruct((2, _N, 16), jnp.float32),
        scratch_types=[
            [pltpu.VMEM((_CHUNK,), jnp.int32) for _ in range(4)],
            pltpu.VMEM((_CHUNK, 16), jnp.float32),
            pltpu.VMEM_SHARED((_N, 16), jnp.float32),
            [pltpu.SemaphoreType.DMA for _ in range(4)],
        ],
        compiler_params=pltpu.CompilerParams(use_tc_tiling_on_sc=False),
    )
    def deg_kernel(dst_hbm, zeros_hbm, ones_hbm, out_hbm, idx, ones_v, acc, sd):
        g = lax.axis_index("c")
        s = lax.axis_index("s")
        slab = pl.ds(s * _ROWS_PER_TILE, _ROWS_PER_TILE)
        pltpu.sync_copy(zeros_hbm.at[slab], acc.at[slab])
        pltpu.sync_copy(ones_hbm, ones_v)
        plsc.subcore_barrier()

        last = _NCHUNK - 1
        for b in (0, 1):
            pltpu.async_copy(dst_hbm.at[g, s, b], idx[b], sd[b])

        def body(k, carry):
            for u in range(4):
                j = 4 * k + u
                bn = (u + 2) % 4
                jn = jnp.minimum(j + 2, last)
                pltpu.make_async_copy(dst_hbm.at[g, s, j], idx[u], sd[u]).wait()
                pltpu.async_copy(dst_hbm.at[g, s, jn], idx[bn], sd[bn])
                pltpu.sync_copy(ones_v, acc.at[idx[u]], add=True)
            return carry

        lax.fori_loop(0, _NCHUNK // 4, body, 0)
        for b in (0, 1):
            pltpu.make_async_copy(dst_hbm.at[g, s, last], idx[b], sd[b]).wait()
        plsc.subcore_barrier()
        pltpu.sync_copy(acc.at[slab], out_hbm.at[g, slab])

    return deg_kernel(dst_idx, zeros16, ones16)


def _sc_aggregate(t_all, src_off, dst_idx, zerosf, feat):
    @functools.partial(
        pl.kernel, mesh=_sc_mesh(),
        out_type=jax.ShapeDtypeStruct((2, _N, feat), jnp.float32),
        scratch_types=[
            pltpu.VMEM((_NCHUNK, _CHUNK), jnp.int32),
            [pltpu.VMEM((_CHUNK,), jnp.int32) for _ in range(4)],
            [pltpu.VMEM((_CHUNK, feat), jnp.float32) for _ in range(4)],
            pltpu.VMEM_SHARED((_N, feat), jnp.float32),
            [pltpu.SemaphoreType.DMA for _ in range(4)],
            [pltpu.SemaphoreType.DMA for _ in range(4)],
        ],
        compiler_params=pltpu.CompilerParams(use_tc_tiling_on_sc=False),
    )
    def agg_kernel(t_hbm, src_hbm, dst_hbm, zeros_hbm, out_hbm,
                   sidx_all, didx, rows, acc, sg, sd):
        g = lax.axis_index("c")
        s = lax.axis_index("s")
        slab = pl.ds(s * _ROWS_PER_TILE, _ROWS_PER_TILE)
        pltpu.sync_copy(zeros_hbm.at[slab], acc.at[slab])
        pltpu.sync_copy(src_hbm.at[g, s], sidx_all)
        plsc.subcore_barrier()

        last = _NCHUNK - 1
        for b in (0, 1):
            pltpu.async_copy(t_hbm.at[sidx_all.at[b]], rows[b], sg[b])
            pltpu.async_copy(dst_hbm.at[g, s, b], didx[b], sd[b])

        def body(k, carry):
            for u in range(4):
                j = 4 * k + u
                b = u
                bn = (u + 2) % 4
                jn = jnp.minimum(j + 2, last)
                pltpu.make_async_copy(
                    t_hbm.at[sidx_all.at[j]], rows[b], sg[b]).wait()
                pltpu.make_async_copy(
                    dst_hbm.at[g, s, j], didx[b], sd[b]).wait()
                pltpu.async_copy(t_hbm.at[sidx_all.at[jn]], rows[bn], sg[bn])
                pltpu.async_copy(dst_hbm.at[g, s, jn], didx[bn], sd[bn])
                pltpu.sync_copy(rows[b], acc.at[didx[b]], add=True)
            return carry

        lax.fori_loop(0, _NCHUNK // 4, body, 0)
        for b in (0, 1):
            pltpu.make_async_copy(t_hbm.at[sidx_all.at[last]], rows[b], sg[b]).wait()
            pltpu.make_async_copy(dst_hbm.at[g, s, last], didx[b], sd[b]).wait()
        plsc.subcore_barrier()
        pltpu.sync_copy(acc.at[slab], out_hbm.at[g, slab])

    return agg_kernel(t_all, src_off, dst_idx, zerosf)


def kernel(x_q, x_c, W1, b1, W2, b2, W3, b3, ins1, ins2, ins3,
           del1, del2, del3, ot_w, ot_b, edge_index_q, edge_index_c):
    ins_params = (ins1, ins2, ins3)
    del_params = (del1, del2, del3)

    src = jnp.stack([edge_index_q[0], edge_index_c[0]])
    dst = jnp.stack([edge_index_q[1], edge_index_c[1]])
    dst_idx = dst.reshape(2, _NTILE, _NCHUNK, _CHUNK)
    src_off = (src + jnp.array([[0], [_N]], jnp.int32)).reshape(
        2, _NTILE, _NCHUNK, _CHUNK)

    zeros16 = jnp.zeros((_N, 16), jnp.float32)
    ones16 = jnp.ones((_CHUNK, 16), jnp.float32)
    deg = _sc_degree(dst_idx, zeros16, ones16)[:, :, 0] + 1.0
    dis = jax.lax.rsqrt(jnp.maximum(deg, 1e-12))[:, :, None]

    x = jnp.concatenate([x_q, x_c], axis=0)
    disf = dis.reshape(2 * _N, 1)
    mcosts = []
    h = x
    for i, (W, b) in enumerate(((W1, b1), (W2, b2), (W3, b3))):
        t = (h @ W) * disf
        agg = _sc_aggregate(t, src_off, dst_idx,
                            jnp.zeros((_N, W.shape[1]), jnp.float32),
                            W.shape[1])
        f = (agg.reshape(2 * _N, -1) + t) * disf + b
        h = jax.nn.relu(f)
        q = f[:_N].reshape(_B, _NG, -1)
        c = f[_N:].reshape(_B, _NG, -1)
        main = -jnp.einsum('bnd,bmd->bnm', q, c)
        dele = -(q @ del_params[i])
        inse = -(c @ ins_params[i])
        mcosts.append(_lap_mcost(main.reshape(_B, _NG * _NG), dele, inse))

    mcost = jnp.stack(mcosts, axis=1)
    mcost_norm = 2.0 * mcost / jnp.float32(_NG + _NG)
    scores = mcost_norm @ ot_w.T + ot_b
    return jax.nn.sigmoid(scores[:, 0])


# speedup vs baseline: 1.0216x; 1.0216x over previous
import functools

import jax
import jax.numpy as jnp
from jax import lax
from jax.experimental import pallas as pl
from jax.experimental.pallas import tpu as pltpu
from jax.experimental.pallas import tpu_sc as plsc

_B = 128
_NG = 64
_NN = 2 * _NG
_BIGC = 99999.0
_MASKV = 1e15

_LAP_CHUNK = 64


def _first2(mask, iota64, g):
    a = jnp.min(jnp.where(mask != 0, iota64, 64), axis=1, keepdims=True)
    b = jnp.min(jnp.where((mask != 0) & (iota64 != a), iota64, 64),
                axis=1, keepdims=True)
    return a, b


def _lap_body(main_ref, dele_ref, inse_ref, out_ref, work_ref):
    g = main_ref.shape[0]
    work_ref[...] = main_ref[...]
    iota4096 = jax.lax.broadcasted_iota(jnp.int32, (g, 4096), 1)
    mrow = iota4096 >> 6
    mcol = iota4096 & 63
    iota64 = jax.lax.broadcasted_iota(jnp.int32, (g, _NG), 1)

    ones64 = jnp.ones((g, _NG), jnp.int32)
    neg1 = jnp.full((g, 1), -1, jnp.int32)
    dele0 = dele_ref[...]
    inse0 = inse_ref[...]
    init = (jnp.zeros((g, 1), jnp.float32),
            ones64, ones64, ones64, ones64,
            neg1, neg1,
            jnp.zeros((g, 1), jnp.int32))

    def _prefix_sum(x):
        for sh in (1, 2, 4, 8, 16, 32):
            x = x + jnp.concatenate(
                [jnp.zeros((g, sh), jnp.int32), x[:, :_NG - sh]], axis=1)
        return x

    def cond(carry):
        return jnp.any(carry[7] < _NN)

    def step(carry):
        acc, frT, frB, frL, frR, rp, cp, npick = carry
        dele = dele0
        inse = inse0

        work = jnp.where((mrow == rp) | (mcol == cp), _MASKV, work_ref[...])
        work_ref[...] = work
        vm = jnp.min(work, axis=1, keepdims=True)
        km = jnp.min(jnp.where(work == vm, iota4096, 4096),
                     axis=1, keepdims=True)
        fm = ((km >> 6) << 7) + (km & 63)
        de = jnp.where((frT & frR) != 0, dele, _MASKV)
        vd = jnp.min(de, axis=1, keepdims=True)
        rd = jnp.min(jnp.where(de == vd, iota64, 64), axis=1, keepdims=True)
        fd = (rd << 7) + 64 + rd
        ie = jnp.where((frB & frL) != 0, inse, _MASKV)
        vi = jnp.min(ie, axis=1, keepdims=True)
        ji = jnp.min(jnp.where(ie == vi, iota64, 64), axis=1, keepdims=True)
        fi = ((ji + 64) << 7) + ji
        jb0, jb1 = _first2(frB, iota64, g)
        ir0, ir1 = _first2(frR, iota64, g)
        z_ok = (jb0 < 64) & (ir0 < 64)
        vz = jnp.where(z_ok, 0.0, _MASKV)
        fz = ((jb0 + 64) << 7) + 64 + ir0
        rt0, rt1 = _first2(frT, iota64, g)
        useA = (ir1 < 64) | ((ir0 < 64) & (ir0 != rt0))
        rb1 = jnp.where(useA, rt0, rt1)
        ib1 = jnp.where(ir0 != rb1, ir0, ir1)
        b1_ok = (rb1 < 64) & (ib1 < 64)
        vb1 = jnp.where(b1_ok, _BIGC, _MASKV)
        fb1 = (rb1 << 7) + 64 + ib1
        cl0, cl1 = _first2(frL, iota64, g)
        useB = (cl1 < 64) | ((cl0 < 64) & (cl0 != jb0))
        jb2 = jnp.where(useB, jb0, jb1)
        cb2 = jnp.where(cl0 != jb2, cl0, cl1)
        b2_ok = (jb2 < 64) & (cb2 < 64)
        vb2 = jnp.where(b2_ok, _BIGC, _MASKV)
        fb2 = ((jb2 + 64) << 7) + cb2

        best_v, best_f = vm, fm
        for v2, f2 in ((vd, fd), (vi, fi), (vz, fz), (vb1, fb1), (vb2, fb2)):
            take = (v2 < best_v) | ((v2 == best_v) & (f2 < best_f))
            best_v = jnp.where(take, v2, best_v)
            best_f = jnp.where(take, f2, best_f)

        r = best_f >> 7
        c = best_f & 127
        active = npick < _NN
        acc = acc + jnp.where(active, best_v, 0.0)
        batch = (best_f == fz) & (vm > 0.0) & (vd > 0.0) & (vi > 0.0)
        cntB = jnp.sum(frB, axis=1, keepdims=True)
        cntR = jnp.sum(frR, axis=1, keepdims=True)
        cnt = jnp.minimum(cntB, cntR)
        rankB = _prefix_sum(frB)
        rankR = _prefix_sum(frR)
        frB_b = jnp.where(rankB <= cnt, 0, frB)
        frR_b = jnp.where(rankR <= cnt, 0, frR)
        frT = jnp.where(iota64 == r, 0, frT)
        frL = jnp.where(iota64 == c, 0, frL)
        frB = jnp.where(batch, frB_b, jnp.where(iota64 == r - 64, 0, frB))
        frR = jnp.where(batch, frR_b, jnp.where(iota64 == c - 64, 0, frR))
        npick = npick + jnp.where(active, jnp.where(batch, cnt, 1), 0)
        return (acc, frT, frB, frL, frR, r, c, npick)

    acc = jax.lax.while_loop(cond, step, init)[0]
    out_ref[...] = acc


def _lap_mcost(main_flat, dele_diag, inse_diag):
    m = main_flat.shape[0]
    nblk = m // _LAP_CHUNK
    out = pl.pallas_call(
        _lap_body,
        grid=(nblk,),
        in_specs=[pl.BlockSpec((_LAP_CHUNK, _NG * _NG), lambda i: (i, 0)),
                  pl.BlockSpec((_LAP_CHUNK, _NG), lambda i: (i, 0)),
                  pl.BlockSpec((_LAP_CHUNK, _NG), lambda i: (i, 0))],
        out_specs=pl.BlockSpec((_LAP_CHUNK, 1), lambda i: (i, 0)),
        out_shape=jax.ShapeDtypeStruct((m, 1), jnp.float32),
        scratch_shapes=[pltpu.VMEM((_LAP_CHUNK, _NG * _NG), jnp.float32)],
    )(main_flat, dele_diag, inse_diag)
    return out.reshape(m)


_N = 8192
_E = 131072
_NTILE = 16
_ROWS_PER_TILE = _N // _NTILE
_CHUNK = 128
_NCHUNK = _E // _NTILE // _CHUNK


def _sc_mesh():
    return plsc.VectorSubcoreMesh(core_axis_name="c", subcore_axis_name="s")


def _sc_degree(dst_idx, zeros16, ones16):
    @functools.partial(
        pl.kernel, mesh=_sc_mesh(),
        out_type=jax.ShapeDtypeStruct((2, _N, 16), jnp.float32),
        scratch_types=[
            [pltpu.VMEM((_CHUNK,), jnp.int32) for _ in range(4)],
            pltpu.VMEM((_CHUNK, 16), jnp.float32),
            pltpu.VMEM_SHARED((_N, 16), jnp.float32),
            [pltpu.SemaphoreType.DMA for _ in range(4)],
        ],
        compiler_params=pltpu.CompilerParams(use_tc_tiling_on_sc=False),
    )
    def deg_kernel(dst_hbm, zeros_hbm, ones_hbm, out_hbm, idx, ones_v, acc, sd):
        g = lax.axis_index("c")
        s = lax.axis_index("s")
        slab = pl.ds(s * _ROWS_PER_TILE, _ROWS_PER_TILE)
        pltpu.sync_copy(zeros_hbm.at[slab], acc.at[slab])
        pltpu.sync_copy(ones_hbm, ones_v)
        plsc.subcore_barrier()

        last = _NCHUNK - 1
        for b in (0, 1):
            pltpu.async_copy(dst_hbm.at[g, s, b], idx[b], sd[b])

        def body(k, carry):
            for u in range(4):
                j = 4 * k + u
                bn = (u + 2) % 4
                jn = jnp.minimum(j + 2, last)
                pltpu.make_async_copy(dst_hbm.at[g, s, j], idx[u], sd[u]).wait()
                pltpu.async_copy(dst_hbm.at[g, s, jn], idx[bn], sd[bn])
                pltpu.sync_copy(ones_v, acc.at[idx[u]], add=True)
            return carry

        lax.fori_loop(0, _NCHUNK // 4, body, 0)
        for b in (0, 1):
            pltpu.make_async_copy(dst_hbm.at[g, s, last], idx[b], sd[b]).wait()
        plsc.subcore_barrier()
        pltpu.sync_copy(acc.at[slab], out_hbm.at[g, slab])

    return deg_kernel(dst_idx, zeros16, ones16)


def _sc_aggregate(t_all, src_off, dst_idx, zerosf, feat):
    @functools.partial(
        pl.kernel, mesh=_sc_mesh(),
        out_type=jax.ShapeDtypeStruct((2, _N, feat), jnp.float32),
        scratch_types=[
            pltpu.VMEM((_NCHUNK, _CHUNK), jnp.int32),
            [pltpu.VMEM((_CHUNK,), jnp.int32) for _ in range(4)],
            [pltpu.VMEM((_CHUNK, feat), jnp.float32) for _ in range(4)],
            pltpu.VMEM_SHARED((_N, feat), jnp.float32),
            [pltpu.SemaphoreType.DMA for _ in range(4)],
            [pltpu.SemaphoreType.DMA for _ in range(4)],
        ],
        compiler_params=pltpu.CompilerParams(use_tc_tiling_on_sc=False),
    )
    def agg_kernel(t_hbm, src_hbm, dst_hbm, zeros_hbm, out_hbm,
                   sidx_all, didx, rows, acc, sg, sd):
        g = lax.axis_index("c")
        s = lax.axis_index("s")
        slab = pl.ds(s * _ROWS_PER_TILE, _ROWS_PER_TILE)
        pltpu.sync_copy(zeros_hbm.at[slab], acc.at[slab])
        pltpu.sync_copy(src_hbm.at[g, s], sidx_all)
        plsc.subcore_barrier()

        last = _NCHUNK - 1
        for b in (0, 1):
            pltpu.async_copy(t_hbm.at[sidx_all.at[b]], rows[b], sg[b])
            pltpu.async_copy(dst_hbm.at[g, s, b], didx[b], sd[b])

        def body(k, carry):
            for u in range(4):
                j = 4 * k + u
                b = u
                bn = (u + 2) % 4
                jn = jnp.minimum(j + 2, last)
                pltpu.make_async_copy(
                    t_hbm.at[sidx_all.at[j]], rows[b], sg[b]).wait()
                pltpu.make_async_copy(
                    dst_hbm.at[g, s, j], didx[b], sd[b]).wait()
                pltpu.async_copy(t_hbm.at[sidx_all.at[jn]], rows[bn], sg[bn])
                pltpu.async_copy(dst_hbm.at[g, s, jn], didx[bn], sd[bn])
                pltpu.sync_copy(rows[b], acc.at[didx[b]], add=True)
            return carry

        lax.fori_loop(0, _NCHUNK // 4, body, 0)
        for b in (0, 1):
            pltpu.make_async_copy(t_hbm.at[sidx_all.at[last]], rows[b], sg[b]).wait()
            pltpu.make_async_copy(dst_hbm.at[g, s, last], didx[b], sd[b]).wait()
        plsc.subcore_barrier()
        pltpu.sync_copy(acc.at[slab], out_hbm.at[g, slab])

    return agg_kernel(t_all, src_off, dst_idx, zerosf)


def kernel(x_q, x_c, W1, b1, W2, b2, W3, b3, ins1, ins2, ins3,
           del1, del2, del3, ot_w, ot_b, edge_index_q, edge_index_c):
    ins_params = (ins1, ins2, ins3)
    del_params = (del1, del2, del3)

    src = jnp.stack([edge_index_q[0], edge_index_c[0]])
    dst = jnp.stack([edge_index_q[1], edge_index_c[1]])
    dst_idx = dst.reshape(2, _NTILE, _NCHUNK, _CHUNK)
    src_off = (src + jnp.array([[0], [_N]], jnp.int32)).reshape(
        2, _NTILE, _NCHUNK, _CHUNK)

    zeros16 = jnp.zeros((_N, 16), jnp.float32)
    ones16 = jnp.ones((_CHUNK, 16), jnp.float32)
    deg = _sc_degree(dst_idx, zeros16, ones16)[:, :, 0] + 1.0
    dis = jax.lax.rsqrt(jnp.maximum(deg, 1e-12))[:, :, None]

    x = jnp.stack([x_q, x_c])
    mcosts = []
    h = x
    for i, (W, b) in enumerate(((W1, b1), (W2, b2), (W3, b3))):
        t = (h @ W) * dis
        agg = _sc_aggregate(t.reshape(2 * _N, -1), src_off, dst_idx,
                            jnp.zeros((_N, W.shape[1]), jnp.float32),
                            W.shape[1])
        f = (agg + t) * dis + b
        h = jax.nn.relu(f)
        q = f[0].reshape(_B, _NG, -1)
        c = f[1].reshape(_B, _NG, -1)
        main = -jnp.einsum('bnd,bmd->bnm', q, c)
        dele = -(q @ del_params[i])
        inse = -(c @ ins_params[i])
        mcosts.append(_lap_mcost(main.reshape(_B, _NG * _NG), dele, inse))

    mcost = jnp.stack(mcosts, axis=1)
    mcost_norm = 2.0 * mcost / jnp.float32(_NG + _NG)
    scores = mcost_norm @ ot_w.T + ot_b
    return jax.nn.sigmoid(scores[:, 0])
